# baseline (speedup 1.0000x reference)
import functools
import jax
import jax.numpy as jnp
from jax import lax
from jax.experimental import pallas as pl
from jax.experimental.pallas import tpu as pltpu
from jax.experimental.pallas import tpu_sc as plsc

N = 10000
NP = 10240
E = 320000
D = 128
DH = 64
H = 256
C = 5
CP = 16

NC = 2
NS = 16
NW = NC * NS
K = 80
NB = 6
RPT = NP // NS


def _make_sc_agg(width, n_tiles, halves):
    nch = E // (n_tiles * K)
    mesh = plsc.VectorSubcoreMesh(core_axis_name="c", subcore_axis_name="s")
    both = n_tiles == NW
    out_shape = ((NC, halves, NP, width) if both else (1, halves, NP, width))

    @functools.partial(
        pl.kernel,
        mesh=mesh,
        out_type=jax.ShapeDtypeStruct(out_shape, jnp.float32),
        scratch_types=[
            pltpu.VMEM((nch + NB - 1, K), jnp.int32),
            pltpu.VMEM((nch, K), jnp.int32),
            pltpu.VMEM((NB * K, width), jnp.float32),
            pltpu.VMEM_SHARED((NP, width), jnp.float32),
            pltpu.SemaphoreType.DMA(()),
        ],
        compiler_params=pltpu.CompilerParams(use_tc_tiling_on_sc=False),
    )
    def k(xh_hbm, src_hbm, dst_hbm, init_hbm, out_hbm, src_v, dst_v, big, acc,
          gsem):
        cid = lax.axis_index("c")
        sid = lax.axis_index("s")
        wid = sid * NC + cid if both else sid

        def slot(i):
            return big.at[pl.ds((i % NB) * K, K)]

        def g_start(h, i):
            pltpu.make_async_copy(xh_hbm.at[h].at[src_v.at[i]], slot(i),
                                  gsem).start()

        def g_wait(h, i):
            pltpu.make_async_copy(xh_hbm.at[h].at[src_v.at[i]], slot(i),
                                  gsem).wait()

        def work():
            pltpu.sync_copy(src_hbm.at[wid], src_v)
            pltpu.sync_copy(dst_hbm.at[wid], dst_v)

            def half(h, hcarry):
                pltpu.sync_copy(init_hbm.at[pl.ds(sid * RPT, RPT)],
                                acc.at[pl.ds(sid * RPT, RPT)])
                plsc.subcore_barrier()

                def fire(i, carry):
                    g_start(h, i)
                    return carry

                lax.fori_loop(0, NB - 1, fire, 0)

                def body(i, carry):
                    g_wait(h, i)
                    pltpu.sync_copy(slot(i), acc.at[dst_v.at[i]], add=True)
                    g_start(h, i + NB - 1)
                    return carry

                lax.fori_loop(0, nch, body, 0)

                def drain(i, carry):
                    g_wait(h, i)
                    return carry

                lax.fori_loop(nch, nch + NB - 1, drain, 0)
                plsc.subcore_barrier()
                if both:
                    pltpu.sync_copy(acc.at[pl.ds(sid * RPT, RPT)],
                                    out_hbm.at[cid, h, pl.ds(sid * RPT, RPT)])
                else:
                    pltpu.sync_copy(acc.at[pl.ds(sid * RPT, RPT)],
                                    out_hbm.at[0, h, pl.ds(sid * RPT, RPT)])
                return hcarry

            lax.fori_loop(0, halves, half, 0)

        if both:
            work()
        else:
            pl.when(cid == 0)(work)

    return k


def _tc_mlp(partials, W1, b1, W2p):
    BN = 2048

    def body(p_ref, w1_ref, b1_ref, w2_ref, q_ref):
        a = jnp.concatenate(
            [p_ref[0, 0] + p_ref[1, 0], p_ref[0, 1] + p_ref[1, 1]], axis=-1)
        h = jnp.dot(a, w1_ref[...], preferred_element_type=jnp.float32)
        h = jnp.maximum(h + b1_ref[...], 0.0)
        q_ref[...] = jnp.dot(h, w2_ref[...], preferred_element_type=jnp.float32)

    return pl.pallas_call(
        body,
        grid=(NP // BN,),
        in_specs=[
            pl.BlockSpec((NC, 2, BN, DH), lambda i: (0, 0, i, 0)),
            pl.BlockSpec((D, H), lambda i: (0, 0)),
            pl.BlockSpec((1, H), lambda i: (0, 0)),
            pl.BlockSpec((H, CP), lambda i: (0, 0)),
        ],
        out_specs=pl.BlockSpec((BN, CP), lambda i: (i, 0)),
        out_shape=jax.ShapeDtypeStruct((NP, CP), jnp.float32),
    )(partials, W1, b1, W2p)


def _pad_chunks(a, n_tiles, nch):
    a = a.reshape(n_tiles, nch, K)
    pad = jnp.zeros((n_tiles, NB - 1, K), jnp.int32)
    return jnp.concatenate([a, pad], axis=1)


def kernel(x, edge_index, W1, b1, W2, b2):
    src = edge_index[0]
    dst = edge_index[1]

    nch1 = E // (NW * K)
    agg1 = _make_sc_agg(DH, NW, 2)
    xh = jnp.stack([x[:, :DH], x[:, DH:]])
    partials = agg1(xh, _pad_chunks(src, NW, nch1), dst.reshape(NW, nch1, K),
                    jnp.zeros((NP, DH), jnp.float32))

    W2p = jnp.pad(W2, ((0, 0), (0, CP - C)))
    q = _tc_mlp(partials, W1, b1.reshape(1, H), W2p)

    nch2 = E // (NS * K)
    agg2 = _make_sc_agg(CP, NS, 1)
    b2_init = jnp.broadcast_to(jnp.pad(b2, (0, CP - C)), (NP, CP))
    out = agg2(q[None], _pad_chunks(src, NS, nch2), dst.reshape(NS, nch2, K),
               b2_init)
    return out[0, 0, :N, :C]


# speedup vs baseline: 1.6754x; 1.6754x over previous
import functools
import jax
import jax.numpy as jnp
from jax import lax
from jax.experimental import pallas as pl
from jax.experimental.pallas import tpu as pltpu
from jax.experimental.pallas import tpu_sc as plsc

N = 10000
NP = 10240
E = 320000
D = 128
H = 256
C = 5
CP = 16

NC = 2
NS = 16
NW = NC * NS
K = 128
NB = 4
RPT = NP // NS


def _ceil_chunks(ept):
    return -(-ept // K)


NCH1 = _ceil_chunks(E // NW)
NCH2 = _ceil_chunks(E // NS)


def _sc_agg1(x, src3, dst3, zeros_init):
    mesh = plsc.VectorSubcoreMesh(core_axis_name="c", subcore_axis_name="s")

    @functools.partial(
        pl.kernel,
        mesh=mesh,
        out_type=jax.ShapeDtypeStruct((NC, NP, D), jnp.float32),
        scratch_types=[
            pltpu.VMEM((NCH1, K), jnp.int32),
            pltpu.VMEM((NCH1, K), jnp.int32),
            pltpu.VMEM((K, D), jnp.float32),
            pltpu.VMEM_SHARED((NP, D), jnp.float32),
        ],
    )
    def k(x_hbm, src_hbm, dst_hbm, init_hbm, out_hbm, src_v, dst_v, rows_v,
          acc):
        cid = lax.axis_index("c")
        sid = lax.axis_index("s")
        wid = sid * NC + cid
        rows = pl.ds(sid * RPT, RPT)
        pltpu.sync_copy(init_hbm.at[rows], acc.at[rows])
        pltpu.sync_copy(src_hbm.at[wid], src_v)
        pltpu.sync_copy(dst_hbm.at[wid], dst_v)
        plsc.subcore_barrier()

        def body(i, carry):
            pltpu.sync_copy(x_hbm.at[src_v.at[i]], rows_v)
            pltpu.sync_copy(rows_v, acc.at[dst_v.at[i]], add=True)
            return carry

        lax.fori_loop(0, NCH1, body, 0)
        plsc.subcore_barrier()
        pltpu.sync_copy(acc.at[rows], out_hbm.at[cid].at[rows])

    return k(x, src3, dst3, zeros_init)


def _sc_agg2(q, src3, dst3, b2_init):
    mesh = plsc.VectorSubcoreMesh(core_axis_name="c", subcore_axis_name="s")

    @functools.partial(
        pl.kernel,
        mesh=mesh,
        out_type=jax.ShapeDtypeStruct((NP, CP), jnp.float32),
        scratch_types=[
            pltpu.VMEM((NCH2 + NB - 1, K), jnp.int32),
            pltpu.VMEM((NCH2, K), jnp.int32),
            pltpu.VMEM((NB * K, CP), jnp.float32),
            pltpu.VMEM_SHARED((NP, CP), jnp.float32),
            pltpu.SemaphoreType.DMA(()),
        ],
        compiler_params=pltpu.CompilerParams(use_tc_tiling_on_sc=False),
    )
    def k(q_hbm, src_hbm, dst_hbm, init_hbm, out_hbm, src_v, dst_v, big, acc,
          gsem):
        cid = lax.axis_index("c")
        sid = lax.axis_index("s")

        def slot(i):
            return big.at[pl.ds((i % NB) * K, K)]

        def g_start(i):
            pltpu.make_async_copy(q_hbm.at[src_v.at[i]], slot(i), gsem).start()

        def g_wait(i):
            pltpu.make_async_copy(q_hbm.at[src_v.at[i]], slot(i), gsem).wait()

        @pl.when(cid == 0)
        def _():
            rows = pl.ds(sid * RPT, RPT)
            pltpu.sync_copy(init_hbm.at[rows], acc.at[rows])
            pltpu.sync_copy(src_hbm.at[sid], src_v)
            pltpu.sync_copy(dst_hbm.at[sid], dst_v)
            plsc.subcore_barrier()

            def fire(i, carry):
                g_start(i)
                return carry

            lax.fori_loop(0, NB - 1, fire, 0)

            def body(i, carry):
                g_wait(i)
                pltpu.sync_copy(slot(i), acc.at[dst_v.at[i]], add=True)
                g_start(i + NB - 1)
                return carry

            lax.fori_loop(0, NCH2, body, 0)

            def drain(i, carry):
                g_wait(i)
                return carry

            lax.fori_loop(NCH2, NCH2 + NB - 1, drain, 0)
            plsc.subcore_barrier()
            pltpu.sync_copy(acc.at[rows], out_hbm.at[rows])

    return k(q, src3, dst3, b2_init)


def _tc_mlp(partials, W1, b1, W2p):
    BN = 2048

    def body(p_ref, w1_ref, b1_ref, w2_ref, q_ref):
        a = p_ref[0] + p_ref[1]
        h = jnp.dot(a, w1_ref[...], preferred_element_type=jnp.float32)
        h = jnp.maximum(h + b1_ref[...], 0.0)
        q_ref[...] = jnp.dot(h, w2_ref[...], preferred_element_type=jnp.float32)

    return pl.pallas_call(
        body,
        grid=(NP // BN,),
        in_specs=[
            pl.BlockSpec((NC, BN, D), lambda i: (0, i, 0)),
            pl.BlockSpec((D, H), lambda i: (0, 0)),
            pl.BlockSpec((1, H), lambda i: (0, 0)),
            pl.BlockSpec((H, CP), lambda i: (0, 0)),
        ],
        out_specs=pl.BlockSpec((BN, CP), lambda i: (i, 0)),
        out_shape=jax.ShapeDtypeStruct((NP, CP), jnp.float32),
    )(partials, W1, b1, W2p)


def _pad_idx(a, n_tiles, fill, extra):
    ept = E // n_tiles
    nch = _ceil_chunks(ept)
    a = a.reshape(n_tiles, ept)
    a = jnp.pad(a, ((0, 0), (0, nch * K - ept + extra * K)),
                constant_values=fill)
    return a.reshape(n_tiles, nch + extra, K)


def kernel(x, edge_index, W1, b1, W2, b2):
    src = edge_index[0]
    dst = edge_index[1]

    partials = _sc_agg1(x,
                        _pad_idx(src, NW, 0, 0),
                        _pad_idx(dst, NW, NP - 1, 0),
                        jnp.zeros((NP, D), jnp.float32))

    W2p = jnp.pad(W2, ((0, 0), (0, CP - C)))
    q = _tc_mlp(partials, W1, b1.reshape(1, H), W2p)

    b2_init = jnp.broadcast_to(jnp.pad(b2, (0, CP - C)), (NP, CP))
    out = _sc_agg2(q,
                   _pad_idx(src, NS, 0, NB - 1),
                   _pad_idx(dst, NS, NP - 1, 0),
                   b2_init)
    return out[:N, :C]


# speedup vs baseline: 2.1298x; 1.2712x over previous
import functools
import jax
import jax.numpy as jnp
from jax import lax
from jax.experimental import pallas as pl
from jax.experimental.pallas import tpu as pltpu
from jax.experimental.pallas import tpu_sc as plsc

N = 10000
NP = 10240
E = 320000
D = 128
H = 256
C = 5
CP = 16

NC = 2
NS = 16
NW = NC * NS
K = 80
NB = 4
NCH = E // (NW * K)
RPT = NP // NS


def _sc_agg1(x, src3, dst3, zeros_init):
    mesh = plsc.VectorSubcoreMesh(core_axis_name="c", subcore_axis_name="s")

    @functools.partial(
        pl.kernel,
        mesh=mesh,
        out_type=jax.ShapeDtypeStruct((NC, NP, D), jnp.float32),
        scratch_types=[
            pltpu.VMEM((NCH, K), jnp.int32),
            pltpu.VMEM((NCH, K), jnp.int32),
            pltpu.VMEM((K, D), jnp.float32),
            pltpu.VMEM_SHARED((NP, D), jnp.float32),
        ],
    )
    def k(x_hbm, src_hbm, dst_hbm, init_hbm, out_hbm, src_v, dst_v, rows_v,
          acc):
        cid = lax.axis_index("c")
        sid = lax.axis_index("s")
        wid = sid * NC + cid
        rows = pl.ds(sid * RPT, RPT)
        pltpu.sync_copy(init_hbm.at[rows], acc.at[rows])
        pltpu.sync_copy(src_hbm.at[wid], src_v)
        pltpu.sync_copy(dst_hbm.at[wid], dst_v)
        plsc.subcore_barrier()

        def body(i, carry):
            pltpu.sync_copy(x_hbm.at[src_v.at[i]], rows_v)
            pltpu.sync_copy(rows_v, acc.at[dst_v.at[i]], add=True)
            return carry

        lax.fori_loop(0, NCH, body, 0)
        plsc.subcore_barrier()
        pltpu.sync_copy(acc.at[rows], out_hbm.at[cid].at[rows])

    return k(x, src3, dst3, zeros_init)


def _sc_agg2(q, src3, dst3, init2):
    mesh = plsc.VectorSubcoreMesh(core_axis_name="c", subcore_axis_name="s")

    @functools.partial(
        pl.kernel,
        mesh=mesh,
        out_type=jax.ShapeDtypeStruct((NC, NP, CP), jnp.float32),
        scratch_types=[
            pltpu.VMEM((NCH + NB - 1, K), jnp.int32),
            pltpu.VMEM((NCH, K), jnp.int32),
            pltpu.VMEM((NB * K, CP), jnp.float32),
            pltpu.VMEM_SHARED((NP, CP), jnp.float32),
            pltpu.SemaphoreType.DMA(()),
        ],
        compiler_params=pltpu.CompilerParams(use_tc_tiling_on_sc=False),
    )
    def k(q_hbm, src_hbm, dst_hbm, init_hbm, out_hbm, src_v, dst_v, big, acc,
          gsem):
        cid = lax.axis_index("c")
        sid = lax.axis_index("s")
        wid = sid * NC + cid

        def slot(i):
            return big.at[pl.ds((i % NB) * K, K)]

        def g_start(i):
            pltpu.make_async_copy(q_hbm.at[src_v.at[i]], slot(i), gsem).start()

        def g_wait(i):
            pltpu.make_async_copy(q_hbm.at[src_v.at[i]], slot(i), gsem).wait()

        rows = pl.ds(sid * RPT, RPT)
        pltpu.sync_copy(init_hbm.at[cid].at[rows], acc.at[rows])
        pltpu.sync_copy(src_hbm.at[wid], src_v)
        pltpu.sync_copy(dst_hbm.at[wid], dst_v)
        plsc.subcore_barrier()

        def fire(i, carry):
            g_start(i)
            return carry

        lax.fori_loop(0, NB - 1, fire, 0)

        def body(i, carry):
            g_wait(i)
            pltpu.sync_copy(slot(i), acc.at[dst_v.at[i]], add=True)
            g_start(i + NB - 1)
            return carry

        lax.fori_loop(0, NCH, body, 0)

        def drain(i, carry):
            g_wait(i)
            return carry

        lax.fori_loop(NCH, NCH + NB - 1, drain, 0)
        plsc.subcore_barrier()
        pltpu.sync_copy(acc.at[rows], out_hbm.at[cid].at[rows])

    return k(q, src3, dst3, init2)


def _tc_mlp(partials, W1, b1, W2p):
    BN = 2048

    def body(p_ref, w1_ref, b1_ref, w2_ref, q_ref):
        a = p_ref[0] + p_ref[1]
        h = jnp.dot(a, w1_ref[...], preferred_element_type=jnp.float32)
        h = jnp.maximum(h + b1_ref[...], 0.0)
        q_ref[...] = jnp.dot(h, w2_ref[...], preferred_element_type=jnp.float32)

    return pl.pallas_call(
        body,
        grid=(NP // BN,),
        in_specs=[
            pl.BlockSpec((NC, BN, D), lambda i: (0, i, 0)),
            pl.BlockSpec((D, H), lambda i: (0, 0)),
            pl.BlockSpec((1, H), lambda i: (0, 0)),
            pl.BlockSpec((H, CP), lambda i: (0, 0)),
        ],
        out_specs=pl.BlockSpec((BN, CP), lambda i: (i, 0)),
        out_shape=jax.ShapeDtypeStruct((NP, CP), jnp.float32),
    )(partials, W1, b1, W2p)


def _tc_combine(partials2):
    def body(p_ref, o_ref):
        o_ref[...] = p_ref[0] + p_ref[1]

    return pl.pallas_call(
        body,
        in_specs=[pl.BlockSpec((NC, NP, CP), lambda: (0, 0, 0))],
        out_specs=pl.BlockSpec((NP, CP), lambda: (0, 0)),
        out_shape=jax.ShapeDtypeStruct((NP, CP), jnp.float32),
    )(partials2)


def _pad_idx(a, fill, extra):
    a = a.reshape(NW, NCH, K)
    if extra:
        pad = jnp.full((NW, extra, K), fill, jnp.int32)
        a = jnp.concatenate([a, pad], axis=1)
    return a


def kernel(x, edge_index, W1, b1, W2, b2):
    src3 = _pad_idx(edge_index[0], 0, 0)
    src3p = _pad_idx(edge_index[0], 0, NB - 1)
    dst3 = _pad_idx(edge_index[1], 0, 0)

    partials = _sc_agg1(x, src3, dst3, jnp.zeros((NP, D), jnp.float32))

    W2p = jnp.pad(W2, ((0, 0), (0, CP - C)))
    q = _tc_mlp(partials, W1, b1.reshape(1, H), W2p)

    b2row = jnp.pad(b2, (0, CP - C))
    init2 = jnp.stack([jnp.broadcast_to(b2row, (NP, CP)),
                       jnp.zeros((NP, CP), jnp.float32)])
    partials2 = _sc_agg2(q, src3p, dst3, init2)
    out = _tc_combine(partials2)
    return out[:N, :C]


# speedup vs baseline: 2.2065x; 1.0360x over previous
import functools
import jax
import jax.numpy as jnp
from jax import lax
from jax.experimental import pallas as pl
from jax.experimental.pallas import tpu as pltpu
from jax.experimental.pallas import tpu_sc as plsc

N = 10000
NP = 10240
E = 320000
D = 128
DH = 64
H = 256
C = 5
CP = 16

NC = 2
NS = 16
NW = NC * NS
K = 80
NB = 4
NCH1 = E // (NS * K)
NCH2 = E // (NW * K)
RPT = NP // NS


def _sc_agg1(xh, src3, dst3, zeros_init):
    mesh = plsc.VectorSubcoreMesh(core_axis_name="c", subcore_axis_name="s")

    @functools.partial(
        pl.kernel,
        mesh=mesh,
        out_type=jax.ShapeDtypeStruct((NC, NP, DH), jnp.float32),
        scratch_types=[
            pltpu.VMEM((NCH1 + NB - 1, K), jnp.int32),
            pltpu.VMEM((NCH1, K), jnp.int32),
            pltpu.VMEM((NB * K, DH), jnp.float32),
            pltpu.VMEM_SHARED((NP, DH), jnp.float32),
            pltpu.SemaphoreType.DMA(()),
            pltpu.SemaphoreType.DMA(()),
        ],
        compiler_params=pltpu.CompilerParams(use_tc_tiling_on_sc=False),
    )
    def k(xh_hbm, src_hbm, dst_hbm, init_hbm, out_hbm, src_v, dst_v, big,
          acc, gsem, ssem):
        cid = lax.axis_index("c")
        sid = lax.axis_index("s")

        def slot(i):
            return big.at[pl.ds((i % NB) * K, K)]

        def g_desc(i):
            return pltpu.make_async_copy(xh_hbm.at[cid].at[src_v.at[i]],
                                         slot(i), gsem)

        def s_desc(i):
            return pltpu.make_async_copy(slot(i), acc.at[dst_v.at[i]], ssem)

        rows = pl.ds(sid * RPT, RPT)
        pltpu.sync_copy(init_hbm.at[rows], acc.at[rows])
        pltpu.sync_copy(src_hbm.at[sid], src_v)
        pltpu.sync_copy(dst_hbm.at[sid], dst_v)
        plsc.subcore_barrier()

        def fire(i, carry):
            g_desc(i).start()
            return carry

        lax.fori_loop(0, NB - 1, fire, 0)

        def body(i, carry):
            g_desc(i).wait()
            s_desc(i).start(add=True)

            @pl.when(i >= 1)
            def _():
                s_desc(i).wait()

            g_desc(i + NB - 1).start()
            return carry

        lax.fori_loop(0, NCH1, body, 0)
        s_desc(NCH1 - 1).wait()

        def drain(i, carry):
            g_desc(i).wait()
            return carry

        lax.fori_loop(NCH1, NCH1 + NB - 1, drain, 0)
        plsc.subcore_barrier()
        pltpu.sync_copy(acc.at[rows], out_hbm.at[cid].at[rows])

    return k(xh, src3, dst3, zeros_init)


def _sc_agg2(q, src3, dst3, init2):
    mesh = plsc.VectorSubcoreMesh(core_axis_name="c", subcore_axis_name="s")

    @functools.partial(
        pl.kernel,
        mesh=mesh,
        out_type=jax.ShapeDtypeStruct((NC, NP, CP), jnp.float32),
        scratch_types=[
            pltpu.VMEM((NCH2 + NB - 1, K), jnp.int32),
            pltpu.VMEM((NCH2, K), jnp.int32),
            pltpu.VMEM((NB * K, CP), jnp.float32),
            pltpu.VMEM_SHARED((NP, CP), jnp.float32),
            pltpu.SemaphoreType.DMA(()),
            pltpu.SemaphoreType.DMA(()),
        ],
        compiler_params=pltpu.CompilerParams(use_tc_tiling_on_sc=False),
    )
    def k(q_hbm, src_hbm, dst_hbm, init_hbm, out_hbm, src_v, dst_v, big, acc,
          gsem, ssem):
        cid = lax.axis_index("c")
        sid = lax.axis_index("s")
        wid = sid * NC + cid

        def slot(i):
            return big.at[pl.ds((i % NB) * K, K)]

        def g_desc(i):
            return pltpu.make_async_copy(q_hbm.at[src_v.at[i]], slot(i), gsem)

        def s_desc(i):
            return pltpu.make_async_copy(slot(i), acc.at[dst_v.at[i]], ssem)

        rows = pl.ds(sid * RPT, RPT)
        pltpu.sync_copy(init_hbm.at[cid].at[rows], acc.at[rows])
        pltpu.sync_copy(src_hbm.at[wid], src_v)
        pltpu.sync_copy(dst_hbm.at[wid], dst_v)
        plsc.subcore_barrier()

        def fire(i, carry):
            g_desc(i).start()
            return carry

        lax.fori_loop(0, NB - 1, fire, 0)

        def body(i, carry):
            g_desc(i).wait()
            s_desc(i).start(add=True)

            @pl.when(i >= 1)
            def _():
                s_desc(i).wait()

            g_desc(i + NB - 1).start()
            return carry

        lax.fori_loop(0, NCH2, body, 0)
        s_desc(NCH2 - 1).wait()

        def drain(i, carry):
            g_desc(i).wait()
            return carry

        lax.fori_loop(NCH2, NCH2 + NB - 1, drain, 0)
        plsc.subcore_barrier()
        pltpu.sync_copy(acc.at[rows], out_hbm.at[cid].at[rows])

    return k(q, src3, dst3, init2)


def _tc_mlp(partials, W1, b1, W2p):
    BN = 2048

    def body(p_ref, w1_ref, b1_ref, w2_ref, q_ref):
        a = jnp.concatenate([p_ref[0], p_ref[1]], axis=-1)
        h = jnp.dot(a, w1_ref[...], preferred_element_type=jnp.float32)
        h = jnp.maximum(h + b1_ref[...], 0.0)
        q_ref[...] = jnp.dot(h, w2_ref[...], preferred_element_type=jnp.float32)

    return pl.pallas_call(
        body,
        grid=(NP // BN,),
        in_specs=[
            pl.BlockSpec((NC, BN, DH), lambda i: (0, i, 0)),
            pl.BlockSpec((D, H), lambda i: (0, 0)),
            pl.BlockSpec((1, H), lambda i: (0, 0)),
            pl.BlockSpec((H, CP), lambda i: (0, 0)),
        ],
        out_specs=pl.BlockSpec((BN, CP), lambda i: (i, 0)),
        out_shape=jax.ShapeDtypeStruct((NP, CP), jnp.float32),
    )(partials, W1, b1, W2p)


def _tc_combine(partials2):
    def body(p_ref, o_ref):
        o_ref[...] = p_ref[0] + p_ref[1]

    return pl.pallas_call(
        body,
        in_specs=[pl.BlockSpec((NC, NP, CP), lambda: (0, 0, 0))],
        out_specs=pl.BlockSpec((NP, CP), lambda: (0, 0)),
        out_shape=jax.ShapeDtypeStruct((NP, CP), jnp.float32),
    )(partials2)


def _chunked(a, n_tiles, fill, extra):
    nch = E // (n_tiles * K)
    a = a.reshape(n_tiles, nch, K)
    if extra:
        pad = jnp.full((n_tiles, extra, K), fill, jnp.int32)
        a = jnp.concatenate([a, pad], axis=1)
    return a


def kernel(x, edge_index, W1, b1, W2, b2):
    src = edge_index[0]
    dst = edge_index[1]

    xp = jnp.pad(x, ((0, NP - N), (0, 0)))
    xh = jnp.stack([xp[:, :DH], xp[:, DH:]])
    partials = _sc_agg1(xh,
                        _chunked(src, NS, 0, NB - 1),
                        _chunked(dst, NS, 0, 0),
                        jnp.zeros((NP, DH), jnp.float32))

    W2p = jnp.pad(W2, ((0, 0), (0, CP - C)))
    q = _tc_mlp(partials, W1, b1.reshape(1, H), W2p)

    b2row = jnp.pad(b2, (0, CP - C))
    init2 = jnp.stack([jnp.broadcast_to(b2row, (NP, CP)),
                       jnp.zeros((NP, CP), jnp.float32)])
    partials2 = _sc_agg2(q,
                         _chunked(src, NW, 0, NB - 1),
                         _chunked(dst, NW, 0, 0),
                         init2)
    out = _tc_combine(partials2)
    return out[:N, :C]


# speedup vs baseline: 2.7031x; 1.2251x over previous
import functools
import jax
import jax.numpy as jnp
from jax import lax
from jax.experimental import pallas as pl
from jax.experimental.pallas import tpu as pltpu
from jax.experimental.pallas import tpu_sc as plsc

N = 10000
NP = 10240
E = 320000
D = 128
DH = 64
H = 256
C = 5
CP = 16

NC = 2
NS = 16
NW = NC * NS
K = 80
NB = 4
NCH1 = E // (NS * K)
NCH2 = E // (NW * K)
RPT = NP // NS


def _sc_agg1(xh, src3, dst3, zeros_init):
    mesh = plsc.VectorSubcoreMesh(core_axis_name="c", subcore_axis_name="s")

    @functools.partial(
        pl.kernel,
        mesh=mesh,
        out_type=jax.ShapeDtypeStruct((NC, NP, DH), jnp.float32),
        scratch_types=[
            pltpu.VMEM((NCH1 + NB - 1, K), jnp.int32),
            pltpu.VMEM((NCH1, K), jnp.int32),
            pltpu.VMEM((NB * K, DH), jnp.float32),
            pltpu.VMEM_SHARED((NP, DH), jnp.float32),
            pltpu.SemaphoreType.DMA(()),
            pltpu.SemaphoreType.DMA(()),
        ],
        compiler_params=pltpu.CompilerParams(use_tc_tiling_on_sc=False),
    )
    def k(xh_hbm, src_hbm, dst_hbm, init_hbm, out_hbm, src_v, dst_v, big,
          acc, gsem, ssem):
        cid = lax.axis_index("c")
        sid = lax.axis_index("s")

        def slot(i):
            return big.at[pl.ds((i % NB) * K, K)]

        def g_desc(i):
            return pltpu.make_async_copy(xh_hbm.at[cid].at[src_v.at[i]],
                                         slot(i), gsem)

        def s_desc(i):
            return pltpu.make_async_copy(slot(i), acc.at[dst_v.at[i]], ssem)

        rows = pl.ds(sid * RPT, RPT)
        pltpu.sync_copy(init_hbm.at[rows], acc.at[rows])
        pltpu.sync_copy(src_hbm.at[sid], src_v)
        pltpu.sync_copy(dst_hbm.at[sid], dst_v)
        plsc.subcore_barrier()

        def fire(i, carry):
            g_desc(i).start()
            return carry

        lax.fori_loop(0, NB - 1, fire, 0)

        def body(i, carry):
            g_desc(i).wait()
            s_desc(i).start(add=True)

            @pl.when(i >= 1)
            def _():
                s_desc(i).wait()

            g_desc(i + NB - 1).start()
            return carry

        lax.fori_loop(0, NCH1, body, 0)
        s_desc(NCH1 - 1).wait()

        def drain(i, carry):
            g_desc(i).wait()
            return carry

        lax.fori_loop(NCH1, NCH1 + NB - 1, drain, 0)
        plsc.subcore_barrier()
        pltpu.sync_copy(acc.at[rows], out_hbm.at[cid].at[rows])

    return k(xh, src3, dst3, zeros_init)


def _sc_agg2(q, src3, dst3, init2):
    mesh = plsc.VectorSubcoreMesh(core_axis_name="c", subcore_axis_name="s")

    @functools.partial(
        pl.kernel,
        mesh=mesh,
        out_type=jax.ShapeDtypeStruct((NC, NP, CP), jnp.float32),
        scratch_types=[
            pltpu.VMEM((NCH2 + NB - 1, K), jnp.int32),
            pltpu.VMEM((NCH2, K), jnp.int32),
            pltpu.VMEM((NB * K, CP), jnp.float32),
            pltpu.VMEM_SHARED((NP, CP), jnp.float32),
            pltpu.VMEM_SHARED((NP, CP), jnp.float32),
            pltpu.SemaphoreType.DMA(()),
            pltpu.SemaphoreType.DMA(()),
        ],
        compiler_params=pltpu.CompilerParams(use_tc_tiling_on_sc=False),
    )
    def k(q_hbm, src_hbm, dst_hbm, init_hbm, out_hbm, src_v, dst_v, big, acc,
          qs, gsem, ssem):
        cid = lax.axis_index("c")
        sid = lax.axis_index("s")
        wid = sid * NC + cid

        def slot(i):
            return big.at[pl.ds((i % NB) * K, K)]

        def g_desc(i):
            return pltpu.make_async_copy(qs.at[src_v.at[i]], slot(i), gsem)

        def s_desc(i):
            return pltpu.make_async_copy(slot(i), acc.at[dst_v.at[i]], ssem)

        rows = pl.ds(sid * RPT, RPT)
        pltpu.sync_copy(q_hbm.at[rows], qs.at[rows])
        pltpu.sync_copy(init_hbm.at[cid].at[rows], acc.at[rows])
        pltpu.sync_copy(src_hbm.at[wid], src_v)
        pltpu.sync_copy(dst_hbm.at[wid], dst_v)
        plsc.subcore_barrier()

        def fire(i, carry):
            g_desc(i).start()
            return carry

        lax.fori_loop(0, NB - 1, fire, 0)

        def body(i, carry):
            g_desc(i).wait()
            s_desc(i).start(add=True)

            @pl.when(i >= 1)
            def _():
                s_desc(i).wait()

            g_desc(i + NB - 1).start()
            return carry

        lax.fori_loop(0, NCH2, body, 0)
        s_desc(NCH2 - 1).wait()

        def drain(i, carry):
            g_desc(i).wait()
            return carry

        lax.fori_loop(NCH2, NCH2 + NB - 1, drain, 0)
        plsc.subcore_barrier()
        pltpu.sync_copy(acc.at[rows], out_hbm.at[cid].at[rows])

    return k(q, src3, dst3, init2)


def _tc_mlp(partials, W1, b1, W2p):
    BN = 2048

    def body(p_ref, w1_ref, b1_ref, w2_ref, q_ref):
        a = jnp.concatenate([p_ref[0], p_ref[1]], axis=-1)
        h = jnp.dot(a, w1_ref[...], preferred_element_type=jnp.float32)
        h = jnp.maximum(h + b1_ref[...], 0.0)
        q_ref[...] = jnp.dot(h, w2_ref[...], preferred_element_type=jnp.float32)

    return pl.pallas_call(
        body,
        grid=(NP // BN,),
        in_specs=[
            pl.BlockSpec((NC, BN, DH), lambda i: (0, i, 0)),
            pl.BlockSpec((D, H), lambda i: (0, 0)),
            pl.BlockSpec((1, H), lambda i: (0, 0)),
            pl.BlockSpec((H, CP), lambda i: (0, 0)),
        ],
        out_specs=pl.BlockSpec((BN, CP), lambda i: (i, 0)),
        out_shape=jax.ShapeDtypeStruct((NP, CP), jnp.float32),
    )(partials, W1, b1, W2p)


def _tc_combine(partials2):
    def body(p_ref, o_ref):
        o_ref[...] = p_ref[0] + p_ref[1]

    return pl.pallas_call(
        body,
        in_specs=[pl.BlockSpec((NC, NP, CP), lambda: (0, 0, 0))],
        out_specs=pl.BlockSpec((NP, CP), lambda: (0, 0)),
        out_shape=jax.ShapeDtypeStruct((NP, CP), jnp.float32),
    )(partials2)


def _chunked(a, n_tiles, fill, extra):
    nch = E // (n_tiles * K)
    a = a.reshape(n_tiles, nch, K)
    if extra:
        pad = jnp.full((n_tiles, extra, K), fill, jnp.int32)
        a = jnp.concatenate([a, pad], axis=1)
    return a


def kernel(x, edge_index, W1, b1, W2, b2):
    src = edge_index[0]
    dst = edge_index[1]

    xp = jnp.pad(x, ((0, NP - N), (0, 0)))
    xh = jnp.stack([xp[:, :DH], xp[:, DH:]])
    partials = _sc_agg1(xh,
                        _chunked(src, NS, 0, NB - 1),
                        _chunked(dst, NS, 0, 0),
                        jnp.zeros((NP, DH), jnp.float32))

    W2p = jnp.pad(W2, ((0, 0), (0, CP - C)))
    q = _tc_mlp(partials, W1, b1.reshape(1, H), W2p)

    b2row = jnp.pad(b2, (0, CP - C))
    init2 = jnp.stack([jnp.broadcast_to(b2row, (NP, CP)),
                       jnp.zeros((NP, CP), jnp.float32)])
    partials2 = _sc_agg2(q,
                         _chunked(src, NW, 0, NB - 1),
                         _chunked(dst, NW, 0, 0),
                         init2)
    out = _tc_combine(partials2)
    return out[:N, :C]
